# baseline (speedup 1.0000x reference)
import functools

import jax
import jax.numpy as jnp
from jax import lax
from jax.experimental import pallas as pl
from jax.experimental.pallas import tpu as pltpu
from jax.experimental.pallas import tpu_sc as plsc

_NC = 2
_NS = 16
_NW = _NC * _NS
_LANES = 16
_NXB = 8
_NPB = 2


@functools.cache
def _build(B, T, D, n_emb_rows):
    assert T % _NW == 0 and D % _LANES == 0
    tpw = T // _NW
    ch = 8 if tpw % 8 == 0 else tpw
    n_ch = tpw // ch
    n_items = n_ch * B
    lanes_per_row = D // _LANES
    assert lanes_per_row & (lanes_per_row - 1) == 0
    row_shift = lanes_per_row.bit_length() - 1

    mesh = plsc.VectorSubcoreMesh(
        core_axis_name="c", subcore_axis_name="s",
        num_cores=_NC, num_subcores=_NS)

    @functools.partial(
        pl.kernel,
        out_type=jax.ShapeDtypeStruct((B, T, D), jnp.float32),
        mesh=mesh,
        scratch_types=(
            [pltpu.VMEM((ch, D), jnp.float32) for _ in range(_NXB)]
            + [pltpu.VMEM((ch, D), jnp.float32) for _ in range(_NPB)]
            + [pltpu.SemaphoreType.DMA for _ in range(2 * _NXB + _NPB)]
        ),
    )
    def sc_add(x_hbm, emb_hbm, out_hbm, *scratch):
        xb = scratch[:_NXB]
        pb = scratch[_NXB:_NXB + _NPB]
        sems = scratch[_NXB + _NPB:]
        sx = sems[:_NXB]
        so = sems[_NXB:2 * _NXB]
        sp = sems[2 * _NXB:]

        wid = lax.axis_index("s") * _NC + lax.axis_index("c")
        t0 = wid * tpw

        def start_x(j):
            c, b = divmod(j, B)
            return pltpu.async_copy(
                x_hbm.at[b, pl.ds(t0 + c * ch, ch), :],
                xb[j % _NXB], sx[j % _NXB])

        def start_pe(c):
            return pltpu.async_copy(
                emb_hbm.at[pl.ds(t0 + c * ch, ch), :],
                pb[c % _NPB], sp[c % _NPB])

        x_in = [None] * n_items
        pe_in = [None] * n_ch
        out_dma = [None] * n_items

        pe_in[0] = start_pe(0)
        for j in range(min(_NXB - 1, n_items)):
            x_in[j] = start_x(j)

        for j in range(n_items):
            c, b = divmod(j, B)
            if b == 0 and c + 1 < n_ch:
                pe_in[c + 1] = start_pe(c + 1)
            jn = j + _NXB - 1
            if jn < n_items:
                if jn - _NXB >= 0:
                    out_dma[jn - _NXB].wait()
                x_in[jn] = start_x(jn)
            x_in[j].wait()
            if b == 0:
                pe_in[c].wait()

            buf = xb[j % _NXB]
            pe = pb[c % _NPB]

            def add(i, buf=buf, pe=pe):
                r = lax.shift_right_logical(i, row_shift)
                start = pl.multiple_of(
                    lax.shift_left(i & (lanes_per_row - 1), 4), _LANES)
                sl = pl.ds(start, _LANES)
                plsc.addupdate(buf.at[r, sl], pe[r, sl])

            plsc.parallel_loop(0, ch * lanes_per_row, 1, unroll=8)(add)
            out_dma[j] = pltpu.async_copy(
                buf, out_hbm.at[b, pl.ds(t0 + c * ch, ch), :], so[j % _NXB])

        for j in range(max(0, n_items - _NXB), n_items):
            out_dma[j].wait()

    return sc_add


def kernel(x, embedding):
    B, T, D = x.shape
    fn = _build(B, T, D, embedding.shape[0])
    return fn(x, embedding)


# speedup vs baseline: 1.0701x; 1.0701x over previous
import functools

import jax
import jax.numpy as jnp
from jax import lax
from jax.experimental import pallas as pl
from jax.experimental.pallas import tpu as pltpu
from jax.experimental.pallas import tpu_sc as plsc

_NC = 2
_NS = 16
_NW = _NC * _NS
_LANES = 16
_NXB = 4
_NPB = 2
_NSB = 2


@functools.cache
def _build(B, T, D, n_emb_rows):
    assert T % _NW == 0 and D % _LANES == 0
    tpw = T // _NW
    ch = 16 if tpw % 16 == 0 else tpw
    n_ch = tpw // ch
    n_items = n_ch * B
    lanes_per_row = D // _LANES
    assert lanes_per_row & (lanes_per_row - 1) == 0
    row_shift = lanes_per_row.bit_length() - 1

    mesh = plsc.VectorSubcoreMesh(
        core_axis_name="c", subcore_axis_name="s",
        num_cores=_NC, num_subcores=_NS)

    @functools.partial(
        pl.kernel,
        out_type=jax.ShapeDtypeStruct((B, T, D), jnp.float32),
        mesh=mesh,
        scratch_types=(
            [pltpu.VMEM((ch, D), jnp.float32) for _ in range(_NXB)]
            + [pltpu.VMEM((ch, D), jnp.float32) for _ in range(_NPB)]
            + [pltpu.VMEM_SHARED((_NS, _NSB, ch, D), jnp.float32)]
            + [pltpu.SemaphoreType.DMA
               for _ in range(_NXB + _NPB + 2 * _NSB)]
        ),
    )
    def sc_add(x_hbm, emb_hbm, out_hbm, *scratch):
        xb = scratch[:_NXB]
        pb = scratch[_NXB:_NXB + _NPB]
        stage = scratch[_NXB + _NPB]
        sems = scratch[_NXB + _NPB + 1:]
        sx = sems[:_NXB]
        sp = sems[_NXB:_NXB + _NPB]
        sl_cp = sems[_NXB + _NPB:_NXB + _NPB + _NSB]
        sh_cp = sems[_NXB + _NPB + _NSB:]

        wid = lax.axis_index("s") * _NC + lax.axis_index("c")
        sid = lax.axis_index("s")
        t0 = wid * tpw

        def start_x(j):
            c, b = divmod(j, B)
            return pltpu.async_copy(
                x_hbm.at[b, pl.ds(t0 + c * ch, ch), :],
                xb[j % _NXB], sx[j % _NXB])

        def start_pe(c):
            return pltpu.async_copy(
                emb_hbm.at[pl.ds(t0 + c * ch, ch), :],
                pb[c % _NPB], sp[c % _NPB])

        def start_store(j):
            c, b = divmod(j, B)
            return pltpu.async_copy(
                stage.at[sid, j % _NSB],
                out_hbm.at[b, pl.ds(t0 + c * ch, ch), :],
                sh_cp[j % _NSB])

        x_in = [None] * n_items
        pe_in = [None] * n_ch
        loc_cp = [None] * n_items
        hbm_cp = [None] * n_items

        pe_in[0] = start_pe(0)
        for j in range(min(_NXB - 1, n_items)):
            x_in[j] = start_x(j)

        for j in range(n_items):
            c, b = divmod(j, B)
            if b == 0 and c + 1 < n_ch:
                pe_in[c + 1] = start_pe(c + 1)
            x_in[j].wait()
            if b == 0:
                pe_in[c].wait()

            buf = xb[j % _NXB]
            pe = pb[c % _NPB]

            def add(i, buf=buf, pe=pe):
                r = lax.shift_right_logical(i, row_shift)
                start = pl.multiple_of(
                    lax.shift_left(i & (lanes_per_row - 1), 4), _LANES)
                sl = pl.ds(start, _LANES)
                plsc.addupdate(buf.at[r, sl], pe[r, sl])

            plsc.parallel_loop(0, ch * lanes_per_row, 1, unroll=8)(add)

            if j - _NSB >= 0:
                hbm_cp[j - _NSB].wait()
            loc_cp[j] = pltpu.async_copy(
                buf, stage.at[sid, j % _NSB], sl_cp[j % _NSB])
            if j > 0:
                loc_cp[j - 1].wait()
                hbm_cp[j - 1] = start_store(j - 1)
            jn = j + _NXB - 1
            if _NXB - 1 <= jn < n_items:
                x_in[jn] = start_x(jn)

        loc_cp[n_items - 1].wait()
        hbm_cp[n_items - 1] = start_store(n_items - 1)
        for j in range(n_items - _NSB, n_items):
            if hbm_cp[j] is not None:
                hbm_cp[j].wait()

    return sc_add


def kernel(x, embedding):
    B, T, D = x.shape
    fn = _build(B, T, D, embedding.shape[0])
    return fn(x, embedding)
